# baseline (speedup 1.0000x reference)
import jax
import jax.numpy as jnp
from jax.experimental import pallas as pl
from jax.experimental.pallas import tpu as pltpu

BLOCK_C = 12800
N_BLOCKS = 8
DEPTH = 4


def _sizes(n_classes):
    return [min(BLOCK_C, n_classes - i * BLOCK_C) for i in range(N_BLOCKS)]


def _cwr_body(x_ref, b_ref, w_hbm, o_hbm, wbuf, obuf, otail, wsems, osems):
    n_classes = w_hbm.shape[0]
    sizes = _sizes(n_classes)

    def w_copy(i):
        slot = i % DEPTH
        return pltpu.make_async_copy(
            w_hbm.at[pl.ds(i * BLOCK_C, sizes[i])],
            wbuf.at[slot, pl.ds(0, sizes[i])],
            wsems.at[slot],
        )

    def o_copies(i):
        oslot = i % 2
        base = i * BLOCK_C
        size = sizes[i]
        w0 = (size // 128) * 128
        copies = []
        if w0:
            copies.append(pltpu.make_async_copy(
                obuf.at[oslot, :, pl.ds(0, w0)],
                o_hbm.at[:, pl.ds(base, w0)],
                osems.at[oslot],
            ))
        if size - w0:
            copies.append(pltpu.make_async_copy(
                otail,
                o_hbm.at[:, pl.ds(base + w0, size - w0)],
                osems.at[oslot],
            ))
        return copies

    for i in range(min(DEPTH, N_BLOCKS)):
        w_copy(i).start()

    x = x_ref[...]
    for i in range(N_BLOCKS):
        slot = i % DEPTH
        oslot = i % 2
        w_copy(i).wait()
        acc = jax.lax.dot_general(
            x,
            wbuf[slot],
            (((1,), (1,)), ((), ())),
            preferred_element_type=jnp.float32,
        )
        if i >= 2:
            for c in o_copies(i - 2):
                c.wait()
        sl = slice(i * BLOCK_C, (i + 1) * BLOCK_C)
        res = acc + b_ref[:, sl]
        obuf[oslot] = res
        w0 = (sizes[i] // 128) * 128
        if sizes[i] - w0:
            otail[...] = res[:, w0:w0 + (sizes[i] - w0)]
        for c in o_copies(i):
            c.start()
        if i + DEPTH < N_BLOCKS:
            w_copy(i + DEPTH).start()

    for i in range(N_BLOCKS - 2, N_BLOCKS):
        for c in o_copies(i):
            c.wait()


@jax.jit
def kernel(x, weight, bias):
    n_classes, in_features = weight.shape
    batch = x.shape[0]
    bias2d = bias.reshape(1, n_classes)
    out = pl.pallas_call(
        _cwr_body,
        grid=(1,),
        in_specs=[
            pl.BlockSpec((batch, in_features), lambda i: (0, 0)),
            pl.BlockSpec((1, BLOCK_C * N_BLOCKS), lambda i: (0, 0)),
            pl.BlockSpec(memory_space=pltpu.MemorySpace.HBM),
        ],
        out_specs=pl.BlockSpec(memory_space=pltpu.MemorySpace.HBM),
        out_shape=jax.ShapeDtypeStruct((batch, n_classes), jnp.float32),
        scratch_shapes=[
            pltpu.MemorySpace.VMEM((DEPTH, BLOCK_C, in_features), jnp.float32),
            pltpu.MemorySpace.VMEM((2, batch, BLOCK_C), jnp.float32),
            pltpu.MemorySpace.VMEM((batch, 32), jnp.float32),
            pltpu.SemaphoreType.DMA((DEPTH,)),
            pltpu.SemaphoreType.DMA((2,)),
        ],
    )(x, bias2d, weight)
    return out


# speedup vs baseline: 1.0664x; 1.0664x over previous
import jax
import jax.numpy as jnp
from jax.experimental import pallas as pl
from jax.experimental.pallas import tpu as pltpu

BLOCK_C = 20096


def _linear_block(x_ref, w_ref, b_ref, o_ref):
    acc = jax.lax.dot_general(
        x_ref[...],
        w_ref[...],
        (((1,), (1,)), ((), ())),
        preferred_element_type=jnp.float32,
    )
    o_ref[...] = acc + b_ref[...]


@jax.jit
def kernel(x, weight, bias):
    n_classes, in_features = weight.shape
    batch = x.shape[0]
    bias2d = bias.reshape(1, n_classes)
    grid = (pl.cdiv(n_classes, BLOCK_C),)
    out = pl.pallas_call(
        _linear_block,
        grid=grid,
        in_specs=[
            pl.BlockSpec((batch, in_features), lambda i: (0, 0)),
            pl.BlockSpec((BLOCK_C, in_features), lambda i: (i, 0)),
            pl.BlockSpec((1, BLOCK_C), lambda i: (0, i)),
        ],
        out_specs=pl.BlockSpec((batch, BLOCK_C), lambda i: (0, i)),
        out_shape=jax.ShapeDtypeStruct((batch, n_classes), jnp.float32),
        compiler_params=pltpu.CompilerParams(
            dimension_semantics=("parallel",),
        ),
    )(x, weight, bias2d)
    return out
